# baseline (speedup 1.0000x reference)
import functools

import jax
import jax.numpy as jnp
from jax import lax
from jax.experimental import pallas as pl
from jax.experimental.pallas import tpu as pltpu
from jax.experimental.pallas import tpu_sc as plsc

_N = 10000
_E = 320000
_D = 128
_B = 8
_NP = 10240
_NTILES = 32
_CHUNK = 128
_TCH = 160
_NCHUNKS = _NTILES * _TCH
_KB = 16
_TPS = _NCHUNKS // 16
_SLOTS = _NTILES * _TCH * _CHUNK
_RPS = _NP // 16
_BLK = 2560

_DN_NT = (((1,), (1,)), ((), ()))
_DN_TN = (((0,), (0,)), ((), ()))


def _dense0_body(x_ref, w0_ref, b0_ref, w1_ref, b1_ref, v0_ref, v1_ref):
    x = x_ref[...]
    v0_ref[...] = lax.dot_general(x, w0_ref[...], _DN_NT,
                                  preferred_element_type=jnp.float32) + b0_ref[...]
    v1_ref[...] = lax.dot_general(x, w1_ref[...], _DN_NT,
                                  preferred_element_type=jnp.float32) + b1_ref[...]


def _dense0(x, w0, b0, w1, b1):
    wspec = pl.BlockSpec((_D, _D), lambda i: (0, 0))
    bspec = pl.BlockSpec((1, _D), lambda i: (0, 0))
    rspec = pl.BlockSpec((_BLK, _D), lambda i: (i, 0))
    return pl.pallas_call(
        _dense0_body,
        grid=(_NP // _BLK,),
        in_specs=[rspec, wspec, bspec, wspec, bspec],
        out_specs=[rspec, rspec],
        out_shape=[jax.ShapeDtypeStruct((_NP, _D), jnp.float32)] * 2,
    )(x, w0, b0, w1, b1)


def _dense1_body(v0p_ref, agg_ref, w0_ref, b0_ref, w1_ref, b1_ref,
                 v0_ref, v1_ref):
    x = jnp.maximum(v0p_ref[...] + agg_ref[0], 0.0)
    v0_ref[...] = lax.dot_general(x, w0_ref[...], _DN_NT,
                                  preferred_element_type=jnp.float32) + b0_ref[...]
    v1_ref[...] = lax.dot_general(x, w1_ref[...], _DN_NT,
                                  preferred_element_type=jnp.float32) + b1_ref[...]


def _dense1(v0p, agg, w0, b0, w1, b1):
    wspec = pl.BlockSpec((_D, _D), lambda i: (0, 0))
    bspec = pl.BlockSpec((1, _D), lambda i: (0, 0))
    rspec = pl.BlockSpec((_BLK, _D), lambda i: (i, 0))
    aspec = pl.BlockSpec((1, _BLK, _D), lambda i: (0, i, 0))
    return pl.pallas_call(
        _dense1_body,
        grid=(_NP // _BLK,),
        in_specs=[rspec, aspec, wspec, bspec, wspec, bspec],
        out_specs=[rspec, rspec],
        out_shape=[jax.ShapeDtypeStruct((_NP, _D), jnp.float32)] * 2,
    )(v0p, agg, w0, b0, w1, b1)


def _head_body(v0_ref, agg_ref, idx_ref, fw1_ref, fb1_ref, fw2_ref, fb2_ref,
               out_ref):
    h = jnp.maximum(v0_ref[...] + agg_ref[0], 0.0)
    idx = idx_ref[...]
    onehot = (idx == lax.broadcasted_iota(jnp.int32, (_NP, _B), 1)
              ).astype(jnp.float32)
    counts = jnp.sum(onehot, axis=0)
    sums = lax.dot_general(onehot, h, _DN_TN,
                           preferred_element_type=jnp.float32)
    avg = sums / jnp.maximum(counts, 1.0)[:, None]
    neg = jnp.float32(-jnp.inf)
    mx = jnp.stack(
        [jnp.max(jnp.where(idx == s, h, neg), axis=0) for s in range(_B)],
        axis=0)
    cat = jnp.concatenate([avg, mx], axis=1)
    z = jnp.maximum(
        lax.dot_general(cat, fw1_ref[...], _DN_NT,
                        preferred_element_type=jnp.float32) + fb1_ref[...], 0.0)
    out_ref[...] = lax.dot_general(z, fw2_ref[...], _DN_NT,
                                   preferred_element_type=jnp.float32) + fb2_ref[...]


def _head(v0, agg, vidx, fw1, fb1, fw2, fb2):
    return pl.pallas_call(
        _head_body,
        out_shape=jax.ShapeDtypeStruct((_B, 10), jnp.float32),
    )(v0, agg, vidx, fw1, fb1, fw2, fb2)


@functools.cache
def _make_agg():
    mesh = plsc.VectorSubcoreMesh(core_axis_name="c", subcore_axis_name="s",
                                  num_cores=2, num_subcores=16)

    @functools.partial(
        pl.kernel,
        mesh=mesh,
        out_type=jax.ShapeDtypeStruct((1, _NP, _D), jnp.float32),
        scratch_types=[
            pltpu.VMEM((_KB, _CHUNK), jnp.int32),
            pltpu.VMEM((_KB, _CHUNK), jnp.int32),
            pltpu.VMEM((_CHUNK, _D), jnp.float32),
            pltpu.VMEM((_CHUNK, _D), jnp.float32),
            pltpu.VMEM_SHARED((_NP, _D), jnp.float32),
            pltpu.SemaphoreType.DMA,
            pltpu.SemaphoreType.DMA,
            pltpu.SemaphoreType.DMA,
            pltpu.SemaphoreType.DMA,
        ],
    )
    def _agg(v1_hbm, gidx_hbm, sidx_hbm, zeros_hbm, out_hbm,
             gidx_v, sidx_v, buf0, buf1, acc, gsem0, gsem1, ssem0, ssem1):
        cid = lax.axis_index("c")
        sid = lax.axis_index("s")
        base = sid * _TPS

        @pl.when(cid == 0)
        def _sc0_work():
            r0 = sid * _RPS
            pltpu.sync_copy(zeros_hbm.at[pl.ds(r0, _RPS)], acc.at[pl.ds(r0, _RPS)])
            plsc.subcore_barrier()

            bufs = (buf0, buf1)
            gsems = (gsem0, gsem1)
            ssems = (ssem0, ssem1)

            def blk_body(bk, carry):
                c0 = base + bk * _KB
                pltpu.sync_copy(gidx_hbm.at[pl.ds(c0, _KB)], gidx_v)
                pltpu.sync_copy(sidx_hbm.at[pl.ds(c0, _KB)], sidx_v)

                dg = [None] * _KB
                ds = [None] * _KB
                dg[0] = pltpu.async_copy(v1_hbm.at[gidx_v.at[0]], bufs[0], gsems[0])
                for j in range(_KB):
                    b = j & 1
                    if j + 1 < _KB:
                        if j >= 1:
                            ds[j - 1].wait()
                        dg[j + 1] = pltpu.async_copy(
                            v1_hbm.at[gidx_v.at[j + 1]], bufs[1 - b], gsems[1 - b])
                    dg[j].wait()
                    ds[j] = pltpu.async_copy(
                        bufs[b], acc.at[sidx_v.at[j]], ssems[b], add=True)
                ds[_KB - 2].wait()
                ds[_KB - 1].wait()
                return carry

            lax.fori_loop(0, _TPS // _KB, blk_body, 0)

            plsc.subcore_barrier()
            pltpu.sync_copy(acc.at[pl.ds(r0, _RPS)],
                            out_hbm.at[0, pl.ds(r0, _RPS)])

    return _agg


def kernel(verts, edges, verts_idx, edges_idx,
           w0_0, b0_0, w1_0, b1_0, w0_1, b0_1, w1_1, b1_1,
           fc1_w, fc1_b, fc2_w, fc2_b):
    pad_rows = _NP - _N
    xp = jnp.pad(verts, ((0, pad_rows), (0, 0)))
    src = edges[:, 0]
    dst = edges[:, 1]
    npad = _SLOTS - 2 * _E
    gidx = jnp.concatenate(
        [dst, src, jnp.zeros((npad,), jnp.int32)]).reshape(_NCHUNKS, _CHUNK)
    dump = _N + (jnp.arange(npad, dtype=jnp.int32) % (_NP - _N))
    sidx = jnp.concatenate([src, dst, dump]).reshape(_NCHUNKS, _CHUNK)
    zeros_np = jnp.zeros((_NP, _D), jnp.float32)
    vidx = jnp.pad(verts_idx, (0, pad_rows), constant_values=_B).reshape(_NP, 1)

    b0_0r = b0_0.reshape(1, _D)
    b1_0r = b1_0.reshape(1, _D)
    b0_1r = b0_1.reshape(1, _D)
    b1_1r = b1_1.reshape(1, _D)
    fb1 = fc1_b.reshape(1, -1)
    fb2 = fc2_b.reshape(1, -1)

    agg = _make_agg()
    v0a, v1a = _dense0(xp, w0_0, b0_0r, w1_0, b1_0r)
    agg0 = agg(v1a, gidx, sidx, zeros_np)
    v0b, v1b = _dense1(v0a, agg0, w0_1, b0_1r, w1_1, b1_1r)
    agg1 = agg(v1b, gidx, sidx, zeros_np)
    return _head(v0b, agg1, vidx, fc1_w, fb1, fc2_w, fb2)


# speedup vs baseline: 1.0658x; 1.0658x over previous
import functools

import jax
import jax.numpy as jnp
from jax import lax
from jax.experimental import pallas as pl
from jax.experimental.pallas import tpu as pltpu
from jax.experimental.pallas import tpu_sc as plsc

_N = 10000
_E = 320000
_D = 128
_B = 8
_NP = 10240
_NTILES = 32
_CHUNK = 128
_TCH = 160
_NCHUNKS = _NTILES * _TCH
_KB = 16
_TPS = _NCHUNKS // 16
_SLOTS = _NTILES * _TCH * _CHUNK
_RPS = _NP // 16
_BLK = 2560

_DN_NT = (((1,), (1,)), ((), ()))
_DN_TN = (((0,), (0,)), ((), ()))


def _dense0_body(x_ref, w0_ref, b0_ref, w1_ref, b1_ref, v0_ref, v1_ref):
    x = x_ref[...]
    v0_ref[...] = lax.dot_general(x, w0_ref[...], _DN_NT,
                                  preferred_element_type=jnp.float32) + b0_ref[...]
    v1_ref[...] = lax.dot_general(x, w1_ref[...], _DN_NT,
                                  preferred_element_type=jnp.float32) + b1_ref[...]


def _dense0(x, w0, b0, w1, b1):
    wspec = pl.BlockSpec((_D, _D), lambda i: (0, 0))
    bspec = pl.BlockSpec((1, _D), lambda i: (0, 0))
    rspec = pl.BlockSpec((_BLK, _D), lambda i: (i, 0))
    return pl.pallas_call(
        _dense0_body,
        grid=(_NP // _BLK,),
        in_specs=[rspec, wspec, bspec, wspec, bspec],
        out_specs=[rspec, rspec],
        out_shape=[jax.ShapeDtypeStruct((_NP, _D), jnp.float32)] * 2,
    )(x, w0, b0, w1, b1)


def _dense1_body(v0p_ref, agg_ref, w0_ref, b0_ref, w1_ref, b1_ref,
                 v0_ref, v1_ref):
    x = jnp.maximum(v0p_ref[...] + agg_ref[0], 0.0)
    v0_ref[...] = lax.dot_general(x, w0_ref[...], _DN_NT,
                                  preferred_element_type=jnp.float32) + b0_ref[...]
    v1_ref[...] = lax.dot_general(x, w1_ref[...], _DN_NT,
                                  preferred_element_type=jnp.float32) + b1_ref[...]


def _dense1(v0p, agg, w0, b0, w1, b1):
    wspec = pl.BlockSpec((_D, _D), lambda i: (0, 0))
    bspec = pl.BlockSpec((1, _D), lambda i: (0, 0))
    rspec = pl.BlockSpec((_BLK, _D), lambda i: (i, 0))
    aspec = pl.BlockSpec((1, _BLK, _D), lambda i: (0, i, 0))
    return pl.pallas_call(
        _dense1_body,
        grid=(_NP // _BLK,),
        in_specs=[rspec, aspec, wspec, bspec, wspec, bspec],
        out_specs=[rspec, rspec],
        out_shape=[jax.ShapeDtypeStruct((_NP, _D), jnp.float32)] * 2,
    )(v0p, agg, w0, b0, w1, b1)


def _head_body(v0_ref, agg_ref, idx_ref, fw1_ref, fb1_ref, fw2_ref, fb2_ref,
               out_ref):
    h = jnp.maximum(v0_ref[...] + agg_ref[0], 0.0)
    idx = idx_ref[...]
    onehot = (idx == lax.broadcasted_iota(jnp.int32, (_NP, _B), 1)
              ).astype(jnp.float32)
    counts = jnp.sum(onehot, axis=0)
    sums = lax.dot_general(onehot, h, _DN_TN,
                           preferred_element_type=jnp.float32)
    avg = sums / jnp.maximum(counts, 1.0)[:, None]
    neg = jnp.float32(-jnp.inf)
    mx = jnp.stack(
        [jnp.max(jnp.where(idx == s, h, neg), axis=0) for s in range(_B)],
        axis=0)
    cat = jnp.concatenate([avg, mx], axis=1)
    z = jnp.maximum(
        lax.dot_general(cat, fw1_ref[...], _DN_NT,
                        preferred_element_type=jnp.float32) + fb1_ref[...], 0.0)
    out_ref[...] = lax.dot_general(z, fw2_ref[...], _DN_NT,
                                   preferred_element_type=jnp.float32) + fb2_ref[...]


def _head(v0, agg, vidx, fw1, fb1, fw2, fb2):
    return pl.pallas_call(
        _head_body,
        out_shape=jax.ShapeDtypeStruct((_B, 10), jnp.float32),
    )(v0, agg, vidx, fw1, fb1, fw2, fb2)


@functools.cache
def _make_agg():
    mesh = plsc.VectorSubcoreMesh(core_axis_name="c", subcore_axis_name="s",
                                  num_cores=2, num_subcores=16)

    @functools.partial(
        pl.kernel,
        mesh=mesh,
        out_type=jax.ShapeDtypeStruct((1, _NP, _D), jnp.float32),
        scratch_types=[
            pltpu.VMEM((_KB, _CHUNK), jnp.int32),
            pltpu.VMEM((_KB, _CHUNK), jnp.int32),
            pltpu.VMEM((_CHUNK, _D), jnp.float32),
            pltpu.VMEM((_CHUNK, _D), jnp.float32),
            pltpu.VMEM_SHARED((_NP, _D), jnp.float32),
            pltpu.SemaphoreType.DMA,
            pltpu.SemaphoreType.DMA,
            pltpu.SemaphoreType.DMA,
            pltpu.SemaphoreType.DMA,
        ],
    )
    def _agg(v1_hbm, gidx_hbm, sidx_hbm, zeros_hbm, out_hbm,
             gidx_v, sidx_v, buf0, buf1, acc, gsem0, gsem1, ssem0, ssem1):
        cid = lax.axis_index("c")
        sid = lax.axis_index("s")
        base = sid * _TPS

        @pl.when(cid == 1)
        def _sc0_work():
            r0 = sid * _RPS
            pltpu.sync_copy(zeros_hbm.at[pl.ds(r0, _RPS)], acc.at[pl.ds(r0, _RPS)])
            plsc.subcore_barrier()

            bufs = (buf0, buf1)
            gsems = (gsem0, gsem1)
            ssems = (ssem0, ssem1)

            def blk_body(bk, carry):
                c0 = base + bk * _KB
                pltpu.sync_copy(gidx_hbm.at[pl.ds(c0, _KB)], gidx_v)
                pltpu.sync_copy(sidx_hbm.at[pl.ds(c0, _KB)], sidx_v)

                dg = [None] * _KB
                ds = [None] * _KB
                dg[0] = pltpu.async_copy(v1_hbm.at[gidx_v.at[0]], bufs[0], gsems[0])
                for j in range(_KB):
                    b = j & 1
                    if j + 1 < _KB:
                        if j >= 1:
                            ds[j - 1].wait()
                        dg[j + 1] = pltpu.async_copy(
                            v1_hbm.at[gidx_v.at[j + 1]], bufs[1 - b], gsems[1 - b])
                    dg[j].wait()
                    ds[j] = pltpu.async_copy(
                        bufs[b], acc.at[sidx_v.at[j]], ssems[b], add=True)
                ds[_KB - 2].wait()
                ds[_KB - 1].wait()
                return carry

            lax.fori_loop(0, _TPS // _KB, blk_body, 0)

            plsc.subcore_barrier()
            pltpu.sync_copy(acc.at[pl.ds(r0, _RPS)],
                            out_hbm.at[0, pl.ds(r0, _RPS)])

    return _agg


def kernel(verts, edges, verts_idx, edges_idx,
           w0_0, b0_0, w1_0, b1_0, w0_1, b0_1, w1_1, b1_1,
           fc1_w, fc1_b, fc2_w, fc2_b):
    pad_rows = _NP - _N
    xp = jnp.pad(verts, ((0, pad_rows), (0, 0)))
    src = edges[:, 0]
    dst = edges[:, 1]
    npad = _SLOTS - 2 * _E
    gidx = jnp.concatenate(
        [dst, src, jnp.zeros((npad,), jnp.int32)]).reshape(_NCHUNKS, _CHUNK)
    dump = _N + (jnp.arange(npad, dtype=jnp.int32) % (_NP - _N))
    sidx = jnp.concatenate([src, dst, dump]).reshape(_NCHUNKS, _CHUNK)
    zeros_np = jnp.zeros((_NP, _D), jnp.float32)
    vidx = jnp.pad(verts_idx, (0, pad_rows), constant_values=_B).reshape(_NP, 1)

    b0_0r = b0_0.reshape(1, _D)
    b1_0r = b1_0.reshape(1, _D)
    b0_1r = b0_1.reshape(1, _D)
    b1_1r = b1_1.reshape(1, _D)
    fb1 = fc1_b.reshape(1, -1)
    fb2 = fc2_b.reshape(1, -1)

    agg = _make_agg()
    v0a, v1a = _dense0(xp, w0_0, b0_0r, w1_0, b1_0r)
    agg0 = agg(v1a, gidx, sidx, zeros_np)
    v0b, v1b = _dense1(v0a, agg0, w0_1, b0_1r, w1_1, b1_1r)
    agg1 = agg(v1b, gidx, sidx, zeros_np)
    return _head(v0b, agg1, vidx, fc1_w, fb1, fc2_w, fb2)


# speedup vs baseline: 1.3634x; 1.2792x over previous
import functools

import jax
import jax.numpy as jnp
from jax import lax
from jax.experimental import pallas as pl
from jax.experimental.pallas import tpu as pltpu
from jax.experimental.pallas import tpu_sc as plsc

_N = 10000
_E = 320000
_D = 128
_B = 8
_NP = 10240
_NTILES = 32
_CHUNK = 128
_TCH = 160
_NCHUNKS = _NTILES * _TCH
_KB = 16
_CA = 256
_CB = 2 * _TCH - _CA
_RPS = _NP // 16
_BLK = 2560

_DN_NT = (((1,), (1,)), ((), ()))
_DN_TN = (((0,), (0,)), ((), ()))


def _dense0_body(x_ref, w0_ref, b0_ref, w1_ref, b1_ref, v0_ref, v1_ref):
    x = x_ref[...]
    v0_ref[...] = lax.dot_general(x, w0_ref[...], _DN_NT,
                                  preferred_element_type=jnp.float32) + b0_ref[...]
    v1_ref[...] = lax.dot_general(x, w1_ref[...], _DN_NT,
                                  preferred_element_type=jnp.float32) + b1_ref[...]


def _dense0(x, w0, b0, w1, b1):
    wspec = pl.BlockSpec((_D, _D), lambda i: (0, 0))
    bspec = pl.BlockSpec((1, _D), lambda i: (0, 0))
    rspec = pl.BlockSpec((_BLK, _D), lambda i: (i, 0))
    return pl.pallas_call(
        _dense0_body,
        grid=(_NP // _BLK,),
        in_specs=[rspec, wspec, bspec, wspec, bspec],
        out_specs=[rspec, rspec],
        out_shape=[jax.ShapeDtypeStruct((_NP, _D), jnp.float32)] * 2,
    )(x, w0, b0, w1, b1)


def _dense1_body(v0p_ref, agg_ref, w0_ref, b0_ref, w1_ref, b1_ref,
                 v0_ref, v1_ref):
    x = jnp.maximum(v0p_ref[...] + agg_ref[0] + agg_ref[1], 0.0)
    v0_ref[...] = lax.dot_general(x, w0_ref[...], _DN_NT,
                                  preferred_element_type=jnp.float32) + b0_ref[...]
    v1_ref[...] = lax.dot_general(x, w1_ref[...], _DN_NT,
                                  preferred_element_type=jnp.float32) + b1_ref[...]


def _dense1(v0p, agg, w0, b0, w1, b1):
    wspec = pl.BlockSpec((_D, _D), lambda i: (0, 0))
    bspec = pl.BlockSpec((1, _D), lambda i: (0, 0))
    rspec = pl.BlockSpec((_BLK, _D), lambda i: (i, 0))
    aspec = pl.BlockSpec((2, _BLK, _D), lambda i: (0, i, 0))
    return pl.pallas_call(
        _dense1_body,
        grid=(_NP // _BLK,),
        in_specs=[rspec, aspec, wspec, bspec, wspec, bspec],
        out_specs=[rspec, rspec],
        out_shape=[jax.ShapeDtypeStruct((_NP, _D), jnp.float32)] * 2,
    )(v0p, agg, w0, b0, w1, b1)


def _head_body(v0_ref, agg_ref, idx_ref, fw1_ref, fb1_ref, fw2_ref, fb2_ref,
               out_ref):
    h = jnp.maximum(v0_ref[...] + agg_ref[0] + agg_ref[1], 0.0)
    idx = idx_ref[...]
    onehot = (idx == lax.broadcasted_iota(jnp.int32, (_NP, _B), 1)
              ).astype(jnp.float32)
    counts = jnp.sum(onehot, axis=0)
    sums = lax.dot_general(onehot, h, _DN_TN,
                           preferred_element_type=jnp.float32)
    avg = sums / jnp.maximum(counts, 1.0)[:, None]
    neg = jnp.float32(-jnp.inf)
    mx = jnp.stack(
        [jnp.max(jnp.where(idx == s, h, neg), axis=0) for s in range(_B)],
        axis=0)
    cat = jnp.concatenate([avg, mx], axis=1)
    z = jnp.maximum(
        lax.dot_general(cat, fw1_ref[...], _DN_NT,
                        preferred_element_type=jnp.float32) + fb1_ref[...], 0.0)
    out_ref[...] = lax.dot_general(z, fw2_ref[...], _DN_NT,
                                   preferred_element_type=jnp.float32) + fb2_ref[...]


def _head(v0, agg, vidx, fw1, fb1, fw2, fb2):
    return pl.pallas_call(
        _head_body,
        out_shape=jax.ShapeDtypeStruct((_B, 10), jnp.float32),
    )(v0, agg, vidx, fw1, fb1, fw2, fb2)


@functools.cache
def _make_agg():
    mesh = plsc.VectorSubcoreMesh(core_axis_name="c", subcore_axis_name="s",
                                  num_cores=2, num_subcores=16)

    @functools.partial(
        pl.kernel,
        mesh=mesh,
        out_type=jax.ShapeDtypeStruct((2, _NP, _D), jnp.float32),
        scratch_types=[
            pltpu.VMEM((_KB, _CHUNK), jnp.int32),
            pltpu.VMEM((_KB, _CHUNK), jnp.int32),
            pltpu.VMEM((_CHUNK, _D), jnp.float32),
            pltpu.VMEM((_CHUNK, _D), jnp.float32),
            pltpu.VMEM_SHARED((_NP, _D), jnp.float32),
            pltpu.SemaphoreType.DMA,
            pltpu.SemaphoreType.DMA,
            pltpu.SemaphoreType.DMA,
            pltpu.SemaphoreType.DMA,
        ],
    )
    def _agg(v1_hbm, gidx_hbm, sidx_hbm, zeros_hbm, out_hbm,
             gidx_v, sidx_v, buf0, buf1, acc, gsem0, gsem1, ssem0, ssem1):
        cid = lax.axis_index("c")
        sid = lax.axis_index("s")
        base = jnp.where(cid == 0, sid * _CA, 16 * _CA + sid * _CB)
        nblk = jnp.where(cid == 0, _CA // _KB, _CB // _KB)

        r0 = sid * _RPS
        pltpu.sync_copy(zeros_hbm.at[pl.ds(r0, _RPS)], acc.at[pl.ds(r0, _RPS)])
        plsc.subcore_barrier()

        bufs = (buf0, buf1)
        gsems = (gsem0, gsem1)
        ssems = (ssem0, ssem1)

        def blk_body(bk, carry):
            c0 = base + bk * _KB
            pltpu.sync_copy(gidx_hbm.at[pl.ds(c0, _KB)], gidx_v)
            pltpu.sync_copy(sidx_hbm.at[pl.ds(c0, _KB)], sidx_v)

            dg = [None] * _KB
            ds = [None] * _KB
            dg[0] = pltpu.async_copy(v1_hbm.at[gidx_v.at[0]], bufs[0], gsems[0])
            for j in range(_KB):
                b = j & 1
                if j + 1 < _KB:
                    if j >= 1:
                        ds[j - 1].wait()
                    dg[j + 1] = pltpu.async_copy(
                        v1_hbm.at[gidx_v.at[j + 1]], bufs[1 - b], gsems[1 - b])
                dg[j].wait()
                ds[j] = pltpu.async_copy(
                    bufs[b], acc.at[sidx_v.at[j]], ssems[b], add=True)
            ds[_KB - 2].wait()
            ds[_KB - 1].wait()
            return carry

        lax.fori_loop(0, nblk, blk_body, 0)

        plsc.subcore_barrier()
        pltpu.sync_copy(acc.at[pl.ds(r0, _RPS)], out_hbm.at[cid, pl.ds(r0, _RPS)])

    return _agg


def kernel(verts, edges, verts_idx, edges_idx,
           w0_0, b0_0, w1_0, b1_0, w0_1, b0_1, w1_1, b1_1,
           fc1_w, fc1_b, fc2_w, fc2_b):
    pad_rows = _NP - _N
    xp = jnp.pad(verts, ((0, pad_rows), (0, 0)))
    src = edges[:, 0]
    dst = edges[:, 1]
    npad = _NCHUNKS * _CHUNK - 2 * _E
    gidx = jnp.concatenate(
        [dst, src, jnp.zeros((npad,), jnp.int32)]).reshape(_NCHUNKS, _CHUNK)
    dump = _N + (jnp.arange(npad, dtype=jnp.int32) % (_NP - _N))
    sidx = jnp.concatenate([src, dst, dump]).reshape(_NCHUNKS, _CHUNK)
    zeros_np = jnp.zeros((_NP, _D), jnp.float32)
    vidx = jnp.pad(verts_idx, (0, pad_rows), constant_values=_B).reshape(_NP, 1)

    b0_0r = b0_0.reshape(1, _D)
    b1_0r = b1_0.reshape(1, _D)
    b0_1r = b0_1.reshape(1, _D)
    b1_1r = b1_1.reshape(1, _D)
    fb1 = fc1_b.reshape(1, -1)
    fb2 = fc2_b.reshape(1, -1)

    agg = _make_agg()
    v0a, v1a = _dense0(xp, w0_0, b0_0r, w1_0, b1_0r)
    agg0 = agg(v1a, gidx, sidx, zeros_np)
    v0b, v1b = _dense1(v0a, agg0, w0_1, b0_1r, w1_1, b1_1r)
    agg1 = agg(v1b, gidx, sidx, zeros_np)
    return _head(v0b, agg1, vidx, fc1_w, fb1, fc2_w, fb2)
